# baseline (speedup 1.0000x reference)
import functools

import jax
import jax.numpy as jnp
from jax import lax
from jax.experimental import pallas as pl
from jax.experimental.pallas import tpu as pltpu
from jax.experimental.pallas import tpu_sc as plsc

T = 2048
H = 1024
F = 768
E = 8
BT = 256
N = 2 * T
NBLK = N // BT + E
NSLOT = NBLK * BT
MBT = 1024
NMB = NSLOT // MBT


def _k1_body(x_ref, gw_ref, row_ids_ref, slot_w_ref, pos01_ref, be_ref,
             posr_scr, v_scr, tri_scr):
    sb = pl.program_id(0)

    @pl.when(sb == 0)
    def _meta():
        eids = jax.lax.broadcasted_iota(jnp.int32, (T, E), 1)
        logits = jax.lax.dot_general(
            x_ref[...], gw_ref[...], (((1,), (1,)), ((), ())),
            preferred_element_type=jnp.float32)
        p = jax.nn.softmax(logits, axis=-1)
        i1 = jnp.argmax(p, axis=-1)
        m1 = jnp.max(p, axis=-1, keepdims=True)
        p2 = jnp.where(eids == i1[:, None], -jnp.inf, p)
        i2 = jnp.argmax(p2, axis=-1)
        m2 = jnp.max(p2, axis=-1, keepdims=True)
        w1 = m1 / (m1 + m2)
        w2 = m2 / (m1 + m2)
        h1 = (eids == i1[:, None]).astype(jnp.float32)
        h2 = (eids == i2[:, None]).astype(jnp.float32)
        c1 = jnp.sum(h1, axis=0, keepdims=True)
        c2 = jnp.sum(h2, axis=0, keepdims=True)
        c = c1 + c2
        ri = jax.lax.broadcasted_iota(jnp.int32, (T, T), 0)
        ci = jax.lax.broadcasted_iota(jnp.int32, (T, T), 1)
        tri_scr[...] = (ci < ri).astype(jnp.bfloat16)
        r1m = jnp.dot(tri_scr[...], h1.astype(jnp.bfloat16),
                      preferred_element_type=jnp.float32)
        r2m = jnp.dot(tri_scr[...], h2.astype(jnp.bfloat16),
                      preferred_element_type=jnp.float32)
        r1 = jnp.sum(r1m * h1, axis=1, keepdims=True)
        r2 = jnp.sum(r2m * h2, axis=1, keepdims=True)
        ci32 = c.astype(jnp.int32)
        nb_e = (ci32 + (BT - 1)) // BT
        nbf = nb_e.astype(jnp.float32)
        cols = [jnp.zeros((1, 1), jnp.float32)]
        run = jnp.zeros((1, 1), jnp.float32)
        for eix in range(1, E):
            run = run + nbf[:, eix - 1:eix]
            cols.append(run)
        bs = jnp.concatenate(cols, axis=1)
        offs = bs * BT
        pos0 = jnp.sum(h1 * offs, axis=1, keepdims=True) + r1
        pos1 = jnp.sum(h2 * (offs + c1), axis=1, keepdims=True) + r2
        pos01_ref[:, 0:1] = pos0.astype(jnp.int32)
        pos01_ref[:, 1:2] = pos1.astype(jnp.int32)
        total = jnp.sum(nbf)
        nbiota = jax.lax.broadcasted_iota(
            jnp.int32, (1, NBLK + 8), 1).astype(jnp.float32)
        bacc = jnp.zeros((1, NBLK + 8), jnp.int32)
        for eix in range(E):
            bacc = bacc + (nbiota >= bs[:, eix:eix + 1]).astype(jnp.int32)
        be = jnp.clip(bacc - 1, 0, E - 1)
        be_ref[...] = be
        be_ref[:, NBLK:NBLK + 1] = total.astype(jnp.int32).reshape(1, 1)
        posr_scr[0:1, :] = jnp.transpose(pos0, (1, 0))
        posr_scr[1:2, :] = jnp.transpose(pos1, (1, 0))
        tio = jax.lax.broadcasted_iota(jnp.int32, (T, 1), 0)
        thi = (tio // 256).astype(jnp.float32)
        tlo = (tio % 256).astype(jnp.float32)
        w1h = w1.astype(jnp.bfloat16)
        w1l = (w1 - w1h.astype(jnp.float32)).astype(jnp.bfloat16)
        w2h = w2.astype(jnp.bfloat16)
        w2l = (w2 - w2h.astype(jnp.float32)).astype(jnp.bfloat16)
        v_scr[:, 0:1] = thi.astype(jnp.bfloat16)
        v_scr[:, 1:2] = tlo.astype(jnp.bfloat16)
        v_scr[:, 2:3] = w1h
        v_scr[:, 3:4] = w1l
        v_scr[:, 4:5] = thi.astype(jnp.bfloat16)
        v_scr[:, 5:6] = tlo.astype(jnp.bfloat16)
        v_scr[:, 6:7] = w2h
        v_scr[:, 7:8] = w2l

    slot_col = (jax.lax.broadcasted_iota(jnp.int32, (MBT, 1), 0)
                + sb * MBT).astype(jnp.float32)
    m1t = (slot_col == posr_scr[0:1, :]).astype(jnp.bfloat16)
    m2t = (slot_col == posr_scr[1:2, :]).astype(jnp.bfloat16)
    a = (jnp.dot(m1t, v_scr[:, 0:4], preferred_element_type=jnp.float32)
         + jnp.dot(m2t, v_scr[:, 4:8], preferred_element_type=jnp.float32))
    row_ids_ref[0] = (a[:, 0:1] * 256.0 + a[:, 1:2]).astype(jnp.int32)
    slot_w_ref[0] = a[:, 2:3] + a[:, 3:4]


def _k1_routing(x, gate_w):
    return pl.pallas_call(
        _k1_body,
        grid=(NMB,),
        in_specs=[
            pl.BlockSpec((T, H), lambda sb: (0, 0)),
            pl.BlockSpec((E, H), lambda sb: (0, 0)),
        ],
        out_specs=[
            pl.BlockSpec((1, MBT, 1), lambda sb: (sb, 0, 0)),
            pl.BlockSpec((1, MBT, 1), lambda sb: (sb, 0, 0)),
            pl.BlockSpec((T, 2), lambda sb: (0, 0)),
            pl.BlockSpec((1, NBLK + 8), lambda sb: (0, 0)),
        ],
        out_shape=[
            jax.ShapeDtypeStruct((NMB, MBT, 1), jnp.int32),
            jax.ShapeDtypeStruct((NMB, MBT, 1), jnp.float32),
            jax.ShapeDtypeStruct((T, 2), jnp.int32),
            jax.ShapeDtypeStruct((1, NBLK + 8), jnp.int32),
        ],
        scratch_shapes=[
            pltpu.VMEM((8, T), jnp.float32),
            pltpu.VMEM((T, 8), jnp.bfloat16),
            pltpu.VMEM((T, T), jnp.bfloat16),
        ],
        compiler_params=pltpu.CompilerParams(
            dimension_semantics=("arbitrary",)),
    )(x, gate_w)


def _k2_gather(row_ids, x16):
    info = plsc.get_sparse_core_info()
    nw = info.num_cores * info.num_subcores
    rows_w = NSLOT // nw
    chunk = rows_w // 2
    mesh = plsc.VectorSubcoreMesh(core_axis_name="c", subcore_axis_name="s")

    @functools.partial(
        pl.kernel, mesh=mesh,
        out_type=jax.ShapeDtypeStruct((NSLOT, H // 2), jnp.int32),
        scratch_types=[
            pltpu.VMEM((2, chunk), jnp.int32),
            pltpu.VMEM((chunk, H // 2), jnp.int32),
            pltpu.VMEM((chunk, H // 2), jnp.int32),
            pltpu.SemaphoreType.DMA,
            pltpu.SemaphoreType.DMA,
        ],
    )
    def k(ids_hbm, x_hbm, xd_hbm, idx_v, rows0_v, rows1_v, sem0, sem1):
        wid = lax.axis_index("s") * info.num_cores + lax.axis_index("c")
        base = wid * rows_w
        bufs = (rows0_v, rows1_v)
        sems = (sem0, sem1)
        cps = []
        for ch in range(2):
            pltpu.sync_copy(ids_hbm.at[pl.ds(base + ch * chunk, chunk)],
                            idx_v.at[ch])
            cps.append(pltpu.async_copy(x_hbm.at[idx_v.at[ch]],
                                        bufs[ch], sems[ch]))
        for ch in range(2):
            cps[ch].wait()
            pltpu.sync_copy(bufs[ch],
                            xd_hbm.at[pl.ds(base + ch * chunk, chunk)])

    return k(row_ids, x16)


def _k3_body(be_ref, xd_ref, gp_ref, up_ref, dp_ref, sw_ref, yd_ref):
    nb = pl.program_id(0)

    @pl.when(nb < be_ref[NBLK])
    def _compute():
        xb = xd_ref[...].astype(jnp.float32)
        g = jnp.dot(xb, gp_ref[0], preferred_element_type=jnp.float32)
        u = jnp.dot(xb, up_ref[0], preferred_element_type=jnp.float32)
        act = (g * jax.lax.logistic(g)) * u
        y = jnp.dot(act, dp_ref[0], preferred_element_type=jnp.float32)
        yd_ref[...] = y * sw_ref[0]


def _k3_grouped(be_flat, x_disp, gate_proj, up_proj, down_proj, slot_w):
    grid_spec = pltpu.PrefetchScalarGridSpec(
        num_scalar_prefetch=1,
        grid=(NBLK,),
        in_specs=[
            pl.BlockSpec((BT, H), lambda nb, be: (nb, 0)),
            pl.BlockSpec((1, H, F), lambda nb, be: (be[nb], 0, 0)),
            pl.BlockSpec((1, H, F), lambda nb, be: (be[nb], 0, 0)),
            pl.BlockSpec((1, F, H), lambda nb, be: (be[nb], 0, 0)),
            pl.BlockSpec((1, BT, 1), lambda nb, be: (nb, 0, 0)),
        ],
        out_specs=pl.BlockSpec((BT, H), lambda nb, be: (nb, 0)),
    )
    return pl.pallas_call(
        _k3_body,
        grid_spec=grid_spec,
        out_shape=jax.ShapeDtypeStruct((NSLOT, H), jnp.float32),
        compiler_params=pltpu.CompilerParams(
            dimension_semantics=("arbitrary",)),
    )(be_flat, x_disp, gate_proj, up_proj, down_proj, slot_w)


def _k4_combine(pos_flat, y_disp):
    info = plsc.get_sparse_core_info()
    nw = info.num_cores * info.num_subcores
    tok_w = T // nw
    tchunk = 16
    nch = tok_w // tchunk
    mesh = plsc.VectorSubcoreMesh(core_axis_name="c", subcore_axis_name="s")

    @functools.partial(
        pl.kernel, mesh=mesh,
        out_type=jax.ShapeDtypeStruct((T, H), jnp.float32),
        scratch_types=[
            pltpu.VMEM((nch, 2 * tchunk), jnp.int32),
            pltpu.VMEM((2 * tchunk, H), jnp.float32),
            pltpu.VMEM((2 * tchunk, H), jnp.float32),
            pltpu.VMEM((tchunk, H), jnp.float32),
            pltpu.SemaphoreType.DMA,
            pltpu.SemaphoreType.DMA,
        ],
    )
    def k(pos_hbm, yd_hbm, out_hbm, idx_v, buf0_v, buf1_v, ob_v,
          sem0, sem1):
        wid = lax.axis_index("s") * info.num_cores + lax.axis_index("c")
        tbase = wid * tok_w
        bufs = (buf0_v, buf1_v)
        sems = (sem0, sem1)
        cps = {}
        for ch in range(nch):
            pltpu.sync_copy(
                pos_hbm.at[pl.ds(2 * (tbase + ch * tchunk), 2 * tchunk)],
                idx_v.at[ch])
        for ch in range(2):
            cps[ch] = pltpu.async_copy(yd_hbm.at[idx_v.at[ch]],
                                       bufs[ch], sems[ch])
        for ch in range(nch):
            t0 = tbase + ch * tchunk
            slot = ch % 2
            cps[ch].wait()
            buf = bufs[slot]

            def body(r, carry):
                for cc in range(H // 16):
                    sl = pl.ds(cc * 16, 16)
                    ob_v[r, sl] = buf[2 * r, sl] + buf[2 * r + 1, sl]
                return carry

            lax.fori_loop(0, tchunk, body, 0)
            pltpu.sync_copy(ob_v, out_hbm.at[pl.ds(t0, tchunk)])
            if ch + 2 < nch:
                cps[ch + 2] = pltpu.async_copy(
                    yd_hbm.at[idx_v.at[ch + 2]], bufs[slot], sems[slot])

    return k(pos_flat, y_disp)


def kernel(hidden_states, gate_w, gate_proj, up_proj, down_proj):
    b, s, h = hidden_states.shape
    x = hidden_states.reshape(-1, h)
    row_ids3, slot_w3, pos01, be2 = _k1_routing(x, gate_w)
    row_ids = row_ids3.reshape(NSLOT)
    slot_w = slot_w3.reshape(NBLK, BT, 1)
    pos_flat = pos01.reshape(2 * T)
    be_flat = be2.reshape(NBLK + 8)
    x16w = jax.lax.bitcast_convert_type(
        x.astype(jnp.bfloat16).reshape(T, H // 2, 2), jnp.int32)
    x_disp = jax.lax.bitcast_convert_type(
        _k2_gather(row_ids, x16w), jnp.bfloat16).reshape(NSLOT, H)
    y_disp = _k3_grouped(be_flat, x_disp, gate_proj, up_proj, down_proj,
                         slot_w)
    out = _k4_combine(pos_flat, y_disp)
    return out.reshape(b, s, h)


# speedup vs baseline: 4.1439x; 4.1439x over previous
import jax
import jax.numpy as jnp
from jax.experimental import pallas as pl
from jax.experimental.pallas import tpu as pltpu

TOPK = 2


def _moe_body(x_ref, gw_ref, gp_ref, up_ref, dp_ref, out_ref, w_scr):
    e = pl.program_id(0)
    tb = pl.program_id(1)
    BT = w_scr.shape[0] // pl.num_programs(1)
    E = gw_ref.shape[0]
    eids = jax.lax.broadcasted_iota(jnp.int32, (BT, E), 1)

    @pl.when(e == 0)
    def _router():
        xb32 = x_ref[pl.ds(tb * BT, BT), :]
        logits = jax.lax.dot_general(
            xb32, gw_ref[...], (((1,), (1,)), ((), ())),
            preferred_element_type=jnp.float32)
        p = jax.nn.softmax(logits, axis=-1)
        i1 = jnp.argmax(p, axis=-1)
        m1 = jnp.max(p, axis=-1, keepdims=True)
        p2 = jnp.where(eids == i1[:, None], -jnp.inf, p)
        i2 = jnp.argmax(p2, axis=-1)
        m2 = jnp.max(p2, axis=-1, keepdims=True)
        w = (jnp.where(eids == i1[:, None], m1, 0.0)
             + jnp.where(eids == i2[:, None], m2, 0.0)) / (m1 + m2)
        w_scr[pl.ds(tb * BT, BT), :] = w

    wcol = jnp.sum(w_scr[pl.ds(tb * BT, BT), :] * (eids == e), axis=-1,
                   keepdims=True)

    xb = x_ref[pl.ds(tb * BT, BT), :]
    g = jnp.dot(xb, gp_ref[0], preferred_element_type=jnp.float32)
    u = jnp.dot(xb, up_ref[0], preferred_element_type=jnp.float32)
    act = (g * jax.nn.sigmoid(g)) * u
    y = jnp.dot(act, dp_ref[0], preferred_element_type=jnp.float32)
    contrib = y * wcol

    @pl.when(e == 0)
    def _init():
        out_ref[pl.ds(tb * BT, BT), :] = contrib

    @pl.when(e != 0)
    def _acc():
        out_ref[pl.ds(tb * BT, BT), :] += contrib


def kernel(hidden_states, gate_w, gate_proj, up_proj, down_proj):
    b, s, h = hidden_states.shape
    x = hidden_states.reshape(-1, h)
    T = x.shape[0]
    E, H, F = gate_proj.shape
    BT = 2048
    TB = T // BT

    out = pl.pallas_call(
        _moe_body,
        grid=(E, TB),
        in_specs=[
            pl.BlockSpec((T, H), lambda e, tb: (0, 0)),
            pl.BlockSpec((E, H), lambda e, tb: (0, 0)),
            pl.BlockSpec((1, H, F), lambda e, tb: (e, 0, 0)),
            pl.BlockSpec((1, H, F), lambda e, tb: (e, 0, 0)),
            pl.BlockSpec((1, F, H), lambda e, tb: (e, 0, 0)),
        ],
        out_specs=pl.BlockSpec((T, H), lambda e, tb: (0, 0)),
        out_shape=jax.ShapeDtypeStruct((T, H), jnp.float32),
        scratch_shapes=[
            pltpu.VMEM((T, E), jnp.float32),
        ],
        compiler_params=pltpu.CompilerParams(
            dimension_semantics=("arbitrary", "arbitrary")),
    )(x, gate_w, gate_proj, up_proj, down_proj)
    return out.reshape(b, s, h)
